# baseline (speedup 1.0000x reference)
import functools

import jax
import jax.numpy as jnp
from jax import lax
from jax.experimental import pallas as pl
from jax.experimental.pallas import tpu as pltpu
from jax.experimental.pallas import tpu_sc as plsc

NC = 2
NS = 16
DW = 16


def _pre_body(x_r, pos_r, wa_r, wb_r, wc_r, wp_r, be0_r, a_r, b_r):
    xb = x_r[...]
    ws = wa_r[...] + wc_r[...]
    wd = wb_r[...] - wc_r[...]
    dn = (((1,), (1,)), ((), ()))
    pproj = lax.dot_general(pos_r[...], wp_r[...], dn,
                            preferred_element_type=jnp.float32)
    a_r[...] = lax.dot_general(xb, ws, dn,
                               preferred_element_type=jnp.float32) + pproj
    b_r[...] = (lax.dot_general(xb, wd, dn, preferred_element_type=jnp.float32)
                - pproj + be0_r[...])


def _pre_call(x, pos, wa, wb, wc, wp, be0, bn):
    n, c = x.shape
    grid = (n // bn,)
    return pl.pallas_call(
        _pre_body,
        grid=grid,
        in_specs=[
            pl.BlockSpec((bn, c), lambda i: (i, 0)),
            pl.BlockSpec((bn, 3), lambda i: (i, 0)),
            pl.BlockSpec((c, c), lambda i: (0, 0)),
            pl.BlockSpec((c, c), lambda i: (0, 0)),
            pl.BlockSpec((c, c), lambda i: (0, 0)),
            pl.BlockSpec((c, 3), lambda i: (0, 0)),
            pl.BlockSpec((1, c), lambda i: (0, 0)),
        ],
        out_specs=[
            pl.BlockSpec((bn, c), lambda i: (i, 0)),
            pl.BlockSpec((bn, c), lambda i: (i, 0)),
        ],
        out_shape=[
            jax.ShapeDtypeStruct((n, c), jnp.float32),
            jax.ShapeDtypeStruct((n, c), jnp.float32),
        ],
    )(x, pos, wa, wb, wc, wp, be0)


def _sc_call(a_nodes, b_nodes, src, dst, npad, chunk):
    n, c = a_nodes.shape
    e = src.shape[0]
    nw = NC * NS
    epw = e // nw
    nch = epw // chunk
    rps = npad // NS
    nfull, rem = divmod(rps, chunk)
    dlen = ((rps + 15) // 16) * 16
    olen = ((chunk + 15) // 16) * 16
    assert epw * nw == e and nch * chunk == epw and chunk % 8 == 0
    assert nch % 8 == 2 and nch >= 10
    assert rps * NS == npad and rps % 8 == 0 and rem % 8 == 0

    mesh = plsc.VectorSubcoreMesh(core_axis_name="c", subcore_axis_name="s")

    @functools.partial(
        pl.kernel,
        out_type=[
            jax.ShapeDtypeStruct((NC, npad, c), jnp.float32),
            jax.ShapeDtypeStruct((NC * npad,), jnp.float32),
        ],
        mesh=mesh,
        scratch_types=[
            pltpu.VMEM_SHARED((npad, c), jnp.float32),
            pltpu.VMEM_SHARED((npad,), jnp.float32),
            [pltpu.VMEM((chunk,), jnp.int32) for _ in range(8)],
            [pltpu.VMEM((chunk,), jnp.int32) for _ in range(8)],
            [pltpu.VMEM((chunk, c), jnp.float32) for _ in range(4)],
            [pltpu.VMEM((chunk, c), jnp.float32) for _ in range(4)],
            pltpu.VMEM((olen,), jnp.float32),
            pltpu.VMEM((dlen,), jnp.float32),
            [pltpu.SemaphoreType.DMA for _ in range(8)],
            [pltpu.SemaphoreType.DMA for _ in range(4)],
            [pltpu.SemaphoreType.DMA for _ in range(4)],
            pltpu.SemaphoreType.DMA,
        ],
    )
    def sc_kernel(a_hbm, b_hbm, src_hbm, dst_hbm,
                  out_hbm, outd_hbm,
                  acc, accd, sidx, didx, ras, rbs,
                  ones_v, dbuf, semi, semg, semm, semd):
        cid = lax.axis_index("c")
        s = lax.axis_index("s")
        w = cid * NS + s
        ebase = w * epw
        zvec = jnp.zeros((16,), jnp.float32)
        onevec = jnp.full((16,), jnp.float32(1.0))
        ra0 = ras[0]

        @plsc.parallel_loop(0, chunk, step=1, unroll=4)
        def _zero_rows(i):
            for j in range(c // 16):
                ra0[i, pl.ds(j * 16, 16)] = zvec

        def drow(i, carry):
            dbuf[pl.ds(i * 16, 16)] = zvec
            return carry

        lax.fori_loop(0, dlen // 16, drow, 0)

        def orow(i, carry):
            ones_v[pl.ds(i * 16, 16)] = onevec
            return carry

        lax.fori_loop(0, olen // 16, orow, 0)

        for k in range(nfull):
            pltpu.sync_copy(ra0, acc.at[pl.ds(s * rps + k * chunk, chunk)])
        if rem:
            pltpu.sync_copy(ra0.at[pl.ds(0, rem)],
                            acc.at[pl.ds(s * rps + nfull * chunk, rem)])
        pltpu.sync_copy(dbuf.at[pl.ds(0, rps)], accd.at[pl.ds(s * rps, rps)])
        plsc.subcore_barrier()

        def idx_start(g, q):
            base = ebase + g * chunk
            pltpu.async_copy(src_hbm.at[pl.ds(base, chunk)], sidx[q], semi[q])
            pltpu.async_copy(dst_hbm.at[pl.ds(base, chunk)], didx[q], semi[q])

        def idx_wait(g, q):
            base = ebase + g * chunk
            pltpu.make_async_copy(src_hbm.at[pl.ds(base, chunk)], sidx[q],
                                  semi[q]).wait()
            pltpu.make_async_copy(dst_hbm.at[pl.ds(base, chunk)], didx[q],
                                  semi[q]).wait()

        def gather_start(q, sl):
            pltpu.async_copy(a_hbm.at[sidx[q]], ras[sl], semg[sl])
            pltpu.async_copy(b_hbm.at[didx[q]], rbs[sl], semg[sl])

        def gather_wait(q, sl):
            pltpu.make_async_copy(a_hbm.at[sidx[q]], ras[sl], semg[sl]).wait()
            pltpu.make_async_copy(b_hbm.at[didx[q]], rbs[sl], semg[sl]).wait()

        def scatter_wait(q, sl):
            pltpu.make_async_copy(ras[sl], acc.at[didx[q]], semm[sl]).wait()
            pltpu.make_async_copy(ones_v.at[pl.ds(0, chunk)],
                                  accd.at[didx[q]], semd).wait()

        def process(q, sl):
            ra, rb = ras[sl], rbs[sl]
            gather_wait(q, sl)

            @plsc.parallel_loop(0, chunk, step=1, unroll=4)
            def _relu_rows(i):
                for j in range(c // 16):
                    jsl = pl.ds(j * 16, 16)
                    ra[i, jsl] = jnp.maximum(ra[i, jsl] + rb[i, jsl],
                                             jnp.float32(0.0))

            pltpu.async_copy(ones_v.at[pl.ds(0, chunk)], accd.at[didx[q]],
                             semd)
            pltpu.async_copy(ra, acc.at[didx[q]], semm[sl], add=True)

        for q in range(6):
            idx_start(q, q)
        for g in range(3):
            idx_wait(g, g)
            gather_start(g, g)

        def oct_body(gg, carry):
            for u in range(8):
                g = 8 * gg + u
                process(u, u % 4)

                @pl.when(g >= 1)
                def _():
                    scatter_wait((u - 1) % 8, (u - 1) % 4)

                @pl.when(g + 6 < nch)
                def _():
                    idx_start(g + 6, (u + 6) % 8)

                @pl.when(g + 3 < nch)
                def _():
                    idx_wait(g + 3, (u + 3) % 8)
                    gather_start((u + 3) % 8, (u + 3) % 4)

            return carry

        lax.fori_loop(0, nch // 8, oct_body, 0)
        for t in range(nch - (nch // 8) * 8):
            g = (nch // 8) * 8 + t
            process(g % 8, g % 4)
            scatter_wait((g - 1) % 8, (g - 1) % 4)
        scatter_wait((nch - 1) % 8, (nch - 1) % 4)

        plsc.subcore_barrier()
        for k in range(nfull):
            row0 = s * rps + k * chunk
            pltpu.sync_copy(acc.at[pl.ds(row0, chunk)], ra0)
            pltpu.sync_copy(ra0, out_hbm.at[cid, pl.ds(row0, chunk)])
        if rem:
            row0 = s * rps + nfull * chunk
            pltpu.sync_copy(acc.at[pl.ds(row0, rem)], ra0.at[pl.ds(0, rem)])
            pltpu.sync_copy(ra0.at[pl.ds(0, rem)],
                            out_hbm.at[cid, pl.ds(row0, rem)])
        pltpu.sync_copy(accd.at[pl.ds(s * rps, rps)], dbuf.at[pl.ds(0, rps)])
        pltpu.sync_copy(dbuf.at[pl.ds(0, rps)],
                        outd_hbm.at[pl.ds(cid * npad + s * rps, rps)])

    return sc_kernel(a_nodes, b_nodes, src, dst)


def _post_body(x_r, s_r, d_r, we1_r, be1_r, wn0_r, bn0_r, wn1_r, bn1_r, o_r):
    c = x_r.shape[1]
    dn = (((1,), (1,)), ((), ()))
    ssum = s_r[0] + s_r[1]
    deg = d_r[0] + d_r[1]
    eagg = lax.dot_general(ssum, we1_r[...], dn,
                           preferred_element_type=jnp.float32) + deg * be1_r[...]
    wn0 = wn0_r[...]
    h1 = (lax.dot_general(x_r[...], wn0[:, :c], dn,
                          preferred_element_type=jnp.float32)
          + lax.dot_general(eagg, wn0[:, c:], dn,
                            preferred_element_type=jnp.float32)
          + bn0_r[...])
    h1 = jnp.maximum(h1, jnp.float32(0.0))
    o_r[...] = lax.dot_general(h1, wn1_r[...], dn,
                               preferred_element_type=jnp.float32) + bn1_r[...]


def _post_call(x, sacc, dacc, we1, be1, wn0, bn0, wn1, bn1, bn):
    n, c = x.shape
    grid = (n // bn,)
    return pl.pallas_call(
        _post_body,
        grid=grid,
        in_specs=[
            pl.BlockSpec((bn, c), lambda i: (i, 0)),
            pl.BlockSpec((NC, bn, c), lambda i: (0, i, 0)),
            pl.BlockSpec((NC, bn, 1), lambda i: (0, i, 0)),
            pl.BlockSpec((c, c), lambda i: (0, 0)),
            pl.BlockSpec((1, c), lambda i: (0, 0)),
            pl.BlockSpec((c, 2 * c), lambda i: (0, 0)),
            pl.BlockSpec((1, c), lambda i: (0, 0)),
            pl.BlockSpec((c, c), lambda i: (0, 0)),
            pl.BlockSpec((1, c), lambda i: (0, 0)),
        ],
        out_specs=pl.BlockSpec((bn, c), lambda i: (i, 0)),
        out_shape=jax.ShapeDtypeStruct((n, c), jnp.float32),
    )(x, sacc, dacc, we1, be1, wn0, bn0, wn1, bn1)


def kernel(x, edge_index, pos, We0, be0, We1, be1, Wn0, bn0, Wn1, bn1):
    n, c = x.shape
    src = edge_index[0].astype(jnp.int32)
    dst = edge_index[1].astype(jnp.int32)
    chunk = 80

    wa = We0[:, :c]
    wb = We0[:, c:2 * c]
    wc = We0[:, 2 * c:3 * c]
    wp = We0[:, 3 * c:]

    a_nodes, b_nodes = _pre_call(x, pos, wa, wb, wc, wp, be0[None, :], bn=1000)

    npad = ((n + 8 * NS - 1) // (8 * NS)) * (8 * NS)
    sacc, dacc = _sc_call(a_nodes, b_nodes, src, dst, npad, chunk=40)
    dacc = dacc.reshape(NC, npad)[:, :, None]

    return _post_call(x, sacc, dacc, We1, be1[None, :], Wn0, bn0[None, :],
                      Wn1, bn1[None, :], bn=1000)


# speedup vs baseline: 1.0007x; 1.0007x over previous
import functools

import jax
import jax.numpy as jnp
from jax import lax
from jax.experimental import pallas as pl
from jax.experimental.pallas import tpu as pltpu
from jax.experimental.pallas import tpu_sc as plsc

NC = 2
NS = 16
DW = 16


def _pre_body(x_r, pos_r, wa_r, wb_r, wc_r, wp_r, be0_r, a_r, b_r):
    xb = x_r[...]
    ws = wa_r[...] + wc_r[...]
    wd = wb_r[...] - wc_r[...]
    dn = (((1,), (1,)), ((), ()))
    pproj = lax.dot_general(pos_r[...], wp_r[...], dn,
                            preferred_element_type=jnp.float32)
    a_r[...] = lax.dot_general(xb, ws, dn,
                               preferred_element_type=jnp.float32) + pproj
    b_r[...] = (lax.dot_general(xb, wd, dn, preferred_element_type=jnp.float32)
                - pproj + be0_r[...])


def _pre_call(x, pos, wa, wb, wc, wp, be0, bn):
    n, c = x.shape
    grid = (n // bn,)
    return pl.pallas_call(
        _pre_body,
        grid=grid,
        in_specs=[
            pl.BlockSpec((bn, c), lambda i: (i, 0)),
            pl.BlockSpec((bn, 3), lambda i: (i, 0)),
            pl.BlockSpec((c, c), lambda i: (0, 0)),
            pl.BlockSpec((c, c), lambda i: (0, 0)),
            pl.BlockSpec((c, c), lambda i: (0, 0)),
            pl.BlockSpec((c, 3), lambda i: (0, 0)),
            pl.BlockSpec((1, c), lambda i: (0, 0)),
        ],
        out_specs=[
            pl.BlockSpec((bn, c), lambda i: (i, 0)),
            pl.BlockSpec((bn, c), lambda i: (i, 0)),
        ],
        out_shape=[
            jax.ShapeDtypeStruct((n, c), jnp.float32),
            jax.ShapeDtypeStruct((n, c), jnp.float32),
        ],
    )(x, pos, wa, wb, wc, wp, be0)


def _sc_call(a_nodes, b_nodes, src, dst, npad, chunk):
    n, c = a_nodes.shape
    e = src.shape[0]
    nw = NC * NS
    epw = e // nw
    nch = epw // chunk
    rps = npad // NS
    nfull, rem = divmod(rps, chunk)
    dlen = ((rps + 15) // 16) * 16
    olen = ((chunk + 15) // 16) * 16
    assert epw * nw == e and nch * chunk == epw and chunk % 8 == 0
    assert nch % 8 == 2 and nch >= 10
    assert rps * NS == npad and rps % 8 == 0 and rem % 8 == 0

    mesh = plsc.VectorSubcoreMesh(core_axis_name="c", subcore_axis_name="s")

    @functools.partial(
        pl.kernel,
        out_type=[
            jax.ShapeDtypeStruct((NC, npad, c), jnp.float32),
            jax.ShapeDtypeStruct((NC * npad,), jnp.float32),
        ],
        mesh=mesh,
        scratch_types=[
            pltpu.VMEM_SHARED((npad, c), jnp.float32),
            pltpu.VMEM_SHARED((npad,), jnp.float32),
            [pltpu.VMEM((chunk,), jnp.int32) for _ in range(8)],
            [pltpu.VMEM((chunk,), jnp.int32) for _ in range(8)],
            [pltpu.VMEM((chunk, c), jnp.float32) for _ in range(4)],
            [pltpu.VMEM((chunk, c), jnp.float32) for _ in range(4)],
            pltpu.VMEM((olen,), jnp.float32),
            pltpu.VMEM((dlen,), jnp.float32),
            [pltpu.SemaphoreType.DMA for _ in range(8)],
            [pltpu.SemaphoreType.DMA for _ in range(4)],
            [pltpu.SemaphoreType.DMA for _ in range(4)],
            [pltpu.SemaphoreType.DMA for _ in range(4)],
        ],
    )
    def sc_kernel(a_hbm, b_hbm, src_hbm, dst_hbm,
                  out_hbm, outd_hbm,
                  acc, accd, sidx, didx, ras, rbs,
                  ones_v, dbuf, semi, semg, semm, semd):
        cid = lax.axis_index("c")
        s = lax.axis_index("s")
        w = cid * NS + s
        ebase = w * epw
        zvec = jnp.zeros((16,), jnp.float32)
        onevec = jnp.full((16,), jnp.float32(1.0))
        ra0 = ras[0]

        @plsc.parallel_loop(0, chunk, step=1, unroll=4)
        def _zero_rows(i):
            for j in range(c // 16):
                ra0[i, pl.ds(j * 16, 16)] = zvec

        def drow(i, carry):
            dbuf[pl.ds(i * 16, 16)] = zvec
            return carry

        lax.fori_loop(0, dlen // 16, drow, 0)

        def orow(i, carry):
            ones_v[pl.ds(i * 16, 16)] = onevec
            return carry

        lax.fori_loop(0, olen // 16, orow, 0)

        for k in range(nfull):
            pltpu.sync_copy(ra0, acc.at[pl.ds(s * rps + k * chunk, chunk)])
        if rem:
            pltpu.sync_copy(ra0.at[pl.ds(0, rem)],
                            acc.at[pl.ds(s * rps + nfull * chunk, rem)])
        pltpu.sync_copy(dbuf.at[pl.ds(0, rps)], accd.at[pl.ds(s * rps, rps)])
        plsc.subcore_barrier()

        def idx_start(g, q):
            base = ebase + g * chunk
            pltpu.async_copy(src_hbm.at[pl.ds(base, chunk)], sidx[q], semi[q])
            pltpu.async_copy(dst_hbm.at[pl.ds(base, chunk)], didx[q], semi[q])

        def idx_wait(g, q):
            base = ebase + g * chunk
            pltpu.make_async_copy(src_hbm.at[pl.ds(base, chunk)], sidx[q],
                                  semi[q]).wait()
            pltpu.make_async_copy(dst_hbm.at[pl.ds(base, chunk)], didx[q],
                                  semi[q]).wait()

        def gather_start(q, sl):
            pltpu.async_copy(a_hbm.at[sidx[q]], ras[sl], semg[sl])
            pltpu.async_copy(b_hbm.at[didx[q]], rbs[sl], semg[sl])

        def gather_wait(q, sl):
            pltpu.make_async_copy(a_hbm.at[sidx[q]], ras[sl], semg[sl]).wait()
            pltpu.make_async_copy(b_hbm.at[didx[q]], rbs[sl], semg[sl]).wait()

        def scatter_wait(q, sl):
            pltpu.make_async_copy(ras[sl], acc.at[didx[q]], semm[sl]).wait()
            pltpu.make_async_copy(ones_v.at[pl.ds(0, chunk)],
                                  accd.at[didx[q]], semd[sl]).wait()

        def process(q, sl):
            ra, rb = ras[sl], rbs[sl]
            gather_wait(q, sl)

            @plsc.parallel_loop(0, chunk, step=1, unroll=4)
            def _relu_rows(i):
                for j in range(c // 16):
                    jsl = pl.ds(j * 16, 16)
                    ra[i, jsl] = jnp.maximum(ra[i, jsl] + rb[i, jsl],
                                             jnp.float32(0.0))

            pltpu.async_copy(ones_v.at[pl.ds(0, chunk)], accd.at[didx[q]],
                             semd[sl])
            pltpu.async_copy(ra, acc.at[didx[q]], semm[sl], add=True)

        for q in range(6):
            idx_start(q, q)
        for g in range(3):
            idx_wait(g, g)
            gather_start(g, g)

        def oct_body(gg, carry):
            for u in range(8):
                g = 8 * gg + u
                process(u, u % 4)

                @pl.when(g >= 1)
                def _():
                    scatter_wait((u - 1) % 8, (u - 1) % 4)

                @pl.when(g + 6 < nch)
                def _():
                    idx_start(g + 6, (u + 6) % 8)

                @pl.when(g + 3 < nch)
                def _():
                    idx_wait(g + 3, (u + 3) % 8)
                    gather_start((u + 3) % 8, (u + 3) % 4)

            return carry

        lax.fori_loop(0, nch // 8, oct_body, 0)
        for t in range(nch - (nch // 8) * 8):
            g = (nch // 8) * 8 + t
            process(g % 8, g % 4)
            scatter_wait((g - 1) % 8, (g - 1) % 4)
        scatter_wait((nch - 1) % 8, (nch - 1) % 4)

        plsc.subcore_barrier()
        for k in range(nfull):
            row0 = s * rps + k * chunk
            pltpu.sync_copy(acc.at[pl.ds(row0, chunk)], ra0)
            pltpu.sync_copy(ra0, out_hbm.at[cid, pl.ds(row0, chunk)])
        if rem:
            row0 = s * rps + nfull * chunk
            pltpu.sync_copy(acc.at[pl.ds(row0, rem)], ra0.at[pl.ds(0, rem)])
            pltpu.sync_copy(ra0.at[pl.ds(0, rem)],
                            out_hbm.at[cid, pl.ds(row0, rem)])
        pltpu.sync_copy(accd.at[pl.ds(s * rps, rps)], dbuf.at[pl.ds(0, rps)])
        pltpu.sync_copy(dbuf.at[pl.ds(0, rps)],
                        outd_hbm.at[pl.ds(cid * npad + s * rps, rps)])

    return sc_kernel(a_nodes, b_nodes, src, dst)


def _post_body(x_r, s_r, d_r, we1_r, be1_r, wn0_r, bn0_r, wn1_r, bn1_r, o_r):
    c = x_r.shape[1]
    dn = (((1,), (1,)), ((), ()))
    ssum = s_r[0] + s_r[1]
    deg = d_r[0] + d_r[1]
    eagg = lax.dot_general(ssum, we1_r[...], dn,
                           preferred_element_type=jnp.float32) + deg * be1_r[...]
    wn0 = wn0_r[...]
    h1 = (lax.dot_general(x_r[...], wn0[:, :c], dn,
                          preferred_element_type=jnp.float32)
          + lax.dot_general(eagg, wn0[:, c:], dn,
                            preferred_element_type=jnp.float32)
          + bn0_r[...])
    h1 = jnp.maximum(h1, jnp.float32(0.0))
    o_r[...] = lax.dot_general(h1, wn1_r[...], dn,
                               preferred_element_type=jnp.float32) + bn1_r[...]


def _post_call(x, sacc, dacc, we1, be1, wn0, bn0, wn1, bn1, bn):
    n, c = x.shape
    grid = (n // bn,)
    return pl.pallas_call(
        _post_body,
        grid=grid,
        in_specs=[
            pl.BlockSpec((bn, c), lambda i: (i, 0)),
            pl.BlockSpec((NC, bn, c), lambda i: (0, i, 0)),
            pl.BlockSpec((NC, bn, 1), lambda i: (0, i, 0)),
            pl.BlockSpec((c, c), lambda i: (0, 0)),
            pl.BlockSpec((1, c), lambda i: (0, 0)),
            pl.BlockSpec((c, 2 * c), lambda i: (0, 0)),
            pl.BlockSpec((1, c), lambda i: (0, 0)),
            pl.BlockSpec((c, c), lambda i: (0, 0)),
            pl.BlockSpec((1, c), lambda i: (0, 0)),
        ],
        out_specs=pl.BlockSpec((bn, c), lambda i: (i, 0)),
        out_shape=jax.ShapeDtypeStruct((n, c), jnp.float32),
    )(x, sacc, dacc, we1, be1, wn0, bn0, wn1, bn1)


def kernel(x, edge_index, pos, We0, be0, We1, be1, Wn0, bn0, Wn1, bn1):
    n, c = x.shape
    src = edge_index[0].astype(jnp.int32)
    dst = edge_index[1].astype(jnp.int32)
    chunk = 80

    wa = We0[:, :c]
    wb = We0[:, c:2 * c]
    wc = We0[:, 2 * c:3 * c]
    wp = We0[:, 3 * c:]

    a_nodes, b_nodes = _pre_call(x, pos, wa, wb, wc, wp, be0[None, :], bn=1000)

    npad = ((n + 8 * NS - 1) // (8 * NS)) * (8 * NS)
    sacc, dacc = _sc_call(a_nodes, b_nodes, src, dst, npad, chunk=40)
    dacc = dacc.reshape(NC, npad)[:, :, None]

    return _post_call(x, sacc, dacc, We1, be1[None, :], Wn0, bn0[None, :],
                      Wn1, bn1[None, :], bn=1000)
